# baseline (speedup 1.0000x reference)
import jax
import jax.numpy as jnp
from jax import lax
from jax.experimental import pallas as pl
from jax.experimental.pallas import tpu as pltpu
from jax.experimental.pallas import tpu_sc as plsc

_B = 4
_S = 2048
_H = 2048
_POS_OFFSET = 2
_NC = 2
_NS = 16
_NW = _NC * _NS
_SPW = _S // _NW
_C = 4
_K = _SPW // _C
_G = _B * _C
_IDXC = _G + 8
_LANES = 16
_VPR = _H // _LANES


def _sc_body(ids_hbm, table_hbm, pos_hbm, out_hbm,
             idx_v, rows3, pos3, gsem, psem, wsem):
    wid = lax.axis_index("s") * _NC + lax.axis_index("c")
    s0 = wid * _SPW

    pltpu.sync_copy(ids_hbm.at[wid], idx_v)

    def gather(k, slot):
        pltpu.async_copy(
            table_hbm.at[idx_v.at[pl.ds(k * _IDXC, _G)]], rows3.at[slot], gsem)

    def pload(k, slot):
        pltpu.async_copy(
            pos_hbm.at[idx_v.at[pl.ds(k * _IDXC + _G, _C)]], pos3.at[slot], psem)

    def wout(k, slot):
        for b in range(_B):
            pltpu.async_copy(
                rows3.at[slot].at[pl.ds(b * _C, _C)],
                out_hbm.at[b, pl.ds(s0 + k * _C, _C)],
                wsem,
            )

    def gwait(slot):
        pltpu.make_async_copy(
            table_hbm.at[pl.ds(0, _G)], rows3.at[slot], gsem).wait()

    def pwait(slot):
        pltpu.make_async_copy(
            pos_hbm.at[pl.ds(0, _C)], pos3.at[slot], psem).wait()

    def wwait(slot):
        for b in range(_B):
            pltpu.make_async_copy(
                rows3.at[slot].at[pl.ds(b * _C, _C)],
                out_hbm.at[b, pl.ds(s0, _C)], wsem).wait()

    def add(slot):
        rowsb = rows3.at[slot]
        posb = pos3.at[slot]

        @plsc.parallel_loop(0, _C * _VPR, unroll=2)
        def _(j):
            r = j // _VPR
            off = (j % _VPR) * _LANES
            pv = posb[r, pl.ds(off, _LANES)]
            for b in range(_B):
                row = b * _C + r
                rowsb[row, pl.ds(off, _LANES)] = (
                    rowsb[row, pl.ds(off, _LANES)] + pv
                )

    gather(0, 0)
    gather(1, 1)
    pload(0, 0)
    pload(1, 1)
    pload(2, 2)

    def body(k, carry):
        s = lax.rem(k, 3)
        s1 = lax.rem(k + 1, 3)

        @pl.when(k >= 2)
        def _():
            wwait(s1)

        @pl.when(jnp.logical_and(1 <= k, k + 1 < _K))
        def _():
            gather(k + 1, s1)

        gwait(s)
        pwait(s)
        add(s)
        wout(k, s)

        @pl.when(k + 3 < _K)
        def _():
            pload(k + 3, s)

        return carry

    lax.fori_loop(0, _K, body, 0)

    wwait((_K - 2) % 3)
    wwait((_K - 1) % 3)


@jax.jit
def _embed(ids_r, token_table, pos_table):
    mesh = plsc.VectorSubcoreMesh(core_axis_name="c", subcore_axis_name="s")
    fn = pl.kernel(
        _sc_body,
        out_type=jax.ShapeDtypeStruct((_B, _S, _H), jnp.float32),
        mesh=mesh,
        scratch_types=[
            pltpu.VMEM((_K * _IDXC,), jnp.int32),
            pltpu.VMEM((3, _G, _H), jnp.float32),
            pltpu.VMEM((3, _C, _H), jnp.float32),
            pltpu.SemaphoreType.DMA,
            pltpu.SemaphoreType.DMA,
            pltpu.SemaphoreType.DMA,
        ],
    )
    return fn(ids_r, token_table, pos_table)


def kernel(token_ids, token_table, pos_table):
    tok = (
        token_ids.reshape(_B, _NW, _K, _C)
        .transpose(1, 2, 0, 3)
        .reshape(_NW, _K, _G)
    )
    pos_idx = (
        jnp.arange(_POS_OFFSET, _POS_OFFSET + _S, dtype=jnp.int32)
        .reshape(_NW, _K, _C)
    )
    pad = jnp.zeros((_NW, _K, _IDXC - _G - _C), dtype=jnp.int32)
    ids_r = jnp.concatenate([tok, pos_idx, pad], axis=-1).reshape(_NW, _K * _IDXC)
    return _embed(ids_r, token_table, pos_table)


# speedup vs baseline: 1.0107x; 1.0107x over previous
import jax
import jax.numpy as jnp
from jax import lax
from jax.experimental import pallas as pl
from jax.experimental.pallas import tpu as pltpu
from jax.experimental.pallas import tpu_sc as plsc

_B = 4
_S = 2048
_H = 2048
_POS_OFFSET = 2
_NC = 2
_NS = 16
_NW = _NC * _NS
_SPW = _S // _NW
_C = 4
_K = _SPW // _C
_G = _B * _C
_IDXC = _G + 8
_LANES = 16
_VPR = _H // _LANES


def _sc_body(ids_hbm, table_hbm, pos_hbm, out_hbm,
             idx_v, rows3, pos3, gsem, psem, wsem):
    wid = lax.axis_index("s") * _NC + lax.axis_index("c")
    s0 = wid * _SPW

    pltpu.sync_copy(ids_hbm.at[wid], idx_v)

    def gather(k, slot):
        pltpu.async_copy(
            table_hbm.at[idx_v.at[pl.ds(k * _IDXC, _G)]], rows3.at[slot], gsem)

    def pload(k, slot):
        pltpu.async_copy(
            pos_hbm.at[idx_v.at[pl.ds(k * _IDXC + _G, _C)]], pos3.at[slot], psem)

    def wout(k, slot):
        for b in range(_B):
            pltpu.async_copy(
                rows3.at[slot].at[pl.ds(b * _C, _C)],
                out_hbm.at[b, pl.ds(s0 + k * _C, _C)],
                wsem,
            )

    def gwait(slot):
        pltpu.make_async_copy(
            table_hbm.at[pl.ds(0, _G)], rows3.at[slot], gsem).wait()

    def pwait(slot):
        pltpu.make_async_copy(
            pos_hbm.at[pl.ds(0, _C)], pos3.at[slot], psem).wait()

    def wwait(slot):
        for b in range(_B):
            pltpu.make_async_copy(
                rows3.at[slot].at[pl.ds(b * _C, _C)],
                out_hbm.at[b, pl.ds(s0, _C)], wsem).wait()

    def add(slot):
        rowsb = rows3.at[slot]
        posb = pos3.at[slot]

        @plsc.parallel_loop(0, _C * _VPR, unroll=4)
        def _(j):
            r = j // _VPR
            off = (j % _VPR) * _LANES
            pv = posb[r, pl.ds(off, _LANES)]
            for b in range(_B):
                row = b * _C + r
                rowsb[row, pl.ds(off, _LANES)] = (
                    rowsb[row, pl.ds(off, _LANES)] + pv
                )

    gather(0, 0)
    gather(1, 1)
    pload(0, 0)
    pload(1, 1)
    pload(2, 2)

    def body(k, carry):
        s = lax.rem(k, 3)
        s1 = lax.rem(k + 1, 3)

        @pl.when(k >= 2)
        def _():
            wwait(s1)

        @pl.when(jnp.logical_and(1 <= k, k + 1 < _K))
        def _():
            gather(k + 1, s1)

        gwait(s)
        pwait(s)
        add(s)
        wout(k, s)

        @pl.when(k + 3 < _K)
        def _():
            pload(k + 3, s)

        return carry

    lax.fori_loop(0, _K, body, 0)

    wwait((_K - 2) % 3)
    wwait((_K - 1) % 3)


@jax.jit
def _embed(ids_r, token_table, pos_table):
    mesh = plsc.VectorSubcoreMesh(core_axis_name="c", subcore_axis_name="s")
    fn = pl.kernel(
        _sc_body,
        out_type=jax.ShapeDtypeStruct((_B, _S, _H), jnp.float32),
        mesh=mesh,
        scratch_types=[
            pltpu.VMEM((_K * _IDXC,), jnp.int32),
            pltpu.VMEM((3, _G, _H), jnp.float32),
            pltpu.VMEM((3, _C, _H), jnp.float32),
            pltpu.SemaphoreType.DMA,
            pltpu.SemaphoreType.DMA,
            pltpu.SemaphoreType.DMA,
        ],
    )
    return fn(ids_r, token_table, pos_table)


def kernel(token_ids, token_table, pos_table):
    tok = (
        token_ids.reshape(_B, _NW, _K, _C)
        .transpose(1, 2, 0, 3)
        .reshape(_NW, _K, _G)
    )
    pos_idx = (
        jnp.arange(_POS_OFFSET, _POS_OFFSET + _S, dtype=jnp.int32)
        .reshape(_NW, _K, _C)
    )
    pad = jnp.zeros((_NW, _K, _IDXC - _G - _C), dtype=jnp.int32)
    ids_r = jnp.concatenate([tok, pos_idx, pad], axis=-1).reshape(_NW, _K * _IDXC)
    return _embed(ids_r, token_table, pos_table)
